# baseline (speedup 1.0000x reference)
import functools

import jax
import jax.numpy as jnp
from jax import lax
from jax.experimental import pallas as pl
from jax.experimental.pallas import tpu as pltpu
from jax.experimental.pallas import tpu_sc as plsc

VOCAB_SIZE = 1_000_000
EMBED_DIM = 64
BATCH = 4096
SEQ_LEN = 200
MAX_WAVELENGTH = 10000.0

NUM_CORES = 2
NUM_SUBCORES = 16
NW = NUM_CORES * NUM_SUBCORES
TPW = BATCH * SEQ_LEN // NW
CHUNK = 128
NCHUNK = TPW // CHUNK
NBUF = 4
LANES = 16


def _pos_encoding():
    position = jnp.arange(SEQ_LEN, dtype=jnp.float32)
    min_freq = 1.0 / MAX_WAVELENGTH
    timescales = jnp.power(
        min_freq,
        (2.0 * (jnp.arange(EMBED_DIM, dtype=jnp.float32) // 2)) / float(EMBED_DIM),
    )
    angles = position[:, None] * timescales[None, :]
    cos_mask = jnp.asarray(jnp.arange(EMBED_DIM) % 2, dtype=jnp.float32)
    sin_mask = 1.0 - cos_mask
    return jnp.sin(angles) * sin_mask + jnp.cos(angles) * cos_mask


_mesh = plsc.VectorSubcoreMesh(core_axis_name="c", subcore_axis_name="s")
_params = pltpu.CompilerParams(use_tc_tiling_on_sc=True, needs_layout_passes=False)


@functools.partial(
    pl.kernel,
    out_type=jax.ShapeDtypeStruct((BATCH * SEQ_LEN, 128), jnp.float32),
    mesh=_mesh,
    compiler_params=_params,
    scratch_types=(
        [pltpu.VMEM((CHUNK, 128), jnp.float32) for _ in range(NBUF)]
        + [
            pltpu.VMEM((NCHUNK, CHUNK), jnp.int32),
            pltpu.VMEM((SEQ_LEN // 2, 128), jnp.float32),
        ]
        + [pltpu.SemaphoreType.DMA] * NBUF
        + [pltpu.SemaphoreType.DMA] * NBUF
    ),
)
def _emb_kernel(x_hbm, table_hbm, pos_hbm, out_hbm, *scratch):
    rows = scratch[:NBUF]
    idx_v = scratch[NBUF]
    pos_v = scratch[NBUF + 1]
    sg = scratch[NBUF + 2 : NBUF + 2 + NBUF]
    so = scratch[NBUF + 2 + NBUF :]

    wid = lax.axis_index("s") * NUM_CORES + lax.axis_index("c")
    base = wid * TPW
    pltpu.sync_copy(x_hbm.at[wid], idx_v)
    pltpu.sync_copy(pos_hbm, pos_v)

    def fire_gather(t, a):
        pltpu.async_copy(table_hbm.at[idx_v.at[t]], rows[a], sg[a])

    def wait_gather(t, a):
        pltpu.make_async_copy(table_hbm.at[idx_v.at[t]], rows[a], sg[a]).wait()

    def wait_out(a):
        pltpu.make_async_copy(rows[a], out_hbm.at[pl.ds(base, CHUNK)], so[a]).wait()

    fire_gather(0, 0)
    fire_gather(1, 1)

    def outer(tt, carry):
        for a in range(NBUF):
            t = tt * NBUF + a

            @pl.when(t >= NBUF)
            def _():
                wait_out(a)

            wait_gather(t, a)
            pbase = lax.rem(t * CHUNK, SEQ_LEN)

            def add_pos(k, c):
                l = pbase + k
                l = lax.select(l >= SEQ_LEN, l - SEQ_LEN, l)
                ph = (l & 1) * EMBED_DIM
                lh = l // 2
                for q in range(EMBED_DIM // LANES):
                    sl = pl.ds(q * LANES, LANES)
                    rows[a][k, sl] = rows[a][k, sl] + pos_v[lh, pl.ds(ph + q * LANES, LANES)]
                return c

            lax.fori_loop(0, CHUNK, add_pos, 0, unroll=4)
            pltpu.async_copy(
                rows[a], out_hbm.at[pl.ds(base + t * CHUNK, CHUNK)], so[a]
            )

            tn = t + 2

            @pl.when(tn < NCHUNK)
            def _():
                fire_gather(tn, (a + 2) % NBUF)

        return carry

    lax.fori_loop(0, NCHUNK // NBUF, outer, 0)

    for a in range(NBUF):
        wait_out(a)


def kernel(x, token_emb_table):
    pos2 = _pos_encoding().reshape(SEQ_LEN // 2, 128)
    split = 524288
    table_p = jnp.concatenate(
        [
            jnp.pad(token_emb_table[:split], ((0, 0), (0, 128 - EMBED_DIM))),
            jnp.pad(token_emb_table[split:], ((0, 0), (0, 128 - EMBED_DIM))),
        ],
        axis=0,
    )
    x_r = x.astype(jnp.int32).reshape(NW, NCHUNK, CHUNK)
    out = _emb_kernel(x_r, table_p, pos2)
    return out[:, :EMBED_DIM].reshape(BATCH, SEQ_LEN, EMBED_DIM)


# speedup vs baseline: 1.1479x; 1.1479x over previous
import functools

import jax
import jax.numpy as jnp
from jax import lax
from jax.experimental import pallas as pl
from jax.experimental.pallas import tpu as pltpu
from jax.experimental.pallas import tpu_sc as plsc

VOCAB_SIZE = 1_000_000
EMBED_DIM = 64
BATCH = 4096
SEQ_LEN = 200
MAX_WAVELENGTH = 10000.0

NUM_CORES = 2
NUM_SUBCORES = 16
NW = NUM_CORES * NUM_SUBCORES
TPW = BATCH * SEQ_LEN // NW
CHUNK = 128
NCHUNK = TPW // CHUNK
NBUF = 4
LANES = 16


def _pos_encoding():
    position = jnp.arange(SEQ_LEN, dtype=jnp.float32)
    min_freq = 1.0 / MAX_WAVELENGTH
    timescales = jnp.power(
        min_freq,
        (2.0 * (jnp.arange(EMBED_DIM, dtype=jnp.float32) // 2)) / float(EMBED_DIM),
    )
    angles = position[:, None] * timescales[None, :]
    cos_mask = jnp.asarray(jnp.arange(EMBED_DIM) % 2, dtype=jnp.float32)
    sin_mask = 1.0 - cos_mask
    return jnp.sin(angles) * sin_mask + jnp.cos(angles) * cos_mask


_mesh = plsc.VectorSubcoreMesh(core_axis_name="c", subcore_axis_name="s")
_params = pltpu.CompilerParams(use_tc_tiling_on_sc=True, needs_layout_passes=False)


@functools.partial(
    pl.kernel,
    out_type=jax.ShapeDtypeStruct((BATCH * SEQ_LEN, 128), jnp.float32),
    mesh=_mesh,
    compiler_params=_params,
    scratch_types=(
        [pltpu.VMEM((CHUNK, 128), jnp.float32) for _ in range(NBUF)]
        + [
            pltpu.VMEM((NCHUNK, CHUNK), jnp.int32),
            pltpu.VMEM((SEQ_LEN // 2, 128), jnp.float32),
        ]
        + [pltpu.SemaphoreType.DMA] * NBUF
        + [pltpu.SemaphoreType.DMA] * NBUF
    ),
)
def _emb_kernel(x_hbm, table_hbm, pos_hbm, out_hbm, *scratch):
    rows = scratch[:NBUF]
    idx_v = scratch[NBUF]
    pos_v = scratch[NBUF + 1]
    sg = scratch[NBUF + 2 : NBUF + 2 + NBUF]
    so = scratch[NBUF + 2 + NBUF :]

    wid = lax.axis_index("s") * NUM_CORES + lax.axis_index("c")
    base = wid * TPW
    pltpu.sync_copy(x_hbm.at[wid], idx_v)
    pltpu.sync_copy(pos_hbm, pos_v)

    def fire_gather(t, a):
        pltpu.async_copy(table_hbm.at[idx_v.at[t]], rows[a], sg[a])

    def wait_gather(t, a):
        pltpu.make_async_copy(table_hbm.at[idx_v.at[t]], rows[a], sg[a]).wait()

    def wait_out(a):
        pltpu.make_async_copy(rows[a], out_hbm.at[pl.ds(base, CHUNK)], so[a]).wait()

    fire_gather(0, 0)
    fire_gather(1, 1)

    def outer(tt, carry):
        for a in range(NBUF):
            t = tt * NBUF + a

            wait_gather(t, a)
            pbase = lax.rem(t * CHUNK, SEQ_LEN)

            def add_pos(k, c):
                l = pbase + k
                l = lax.select(l >= SEQ_LEN, l - SEQ_LEN, l)
                ph = (l & 1) * EMBED_DIM
                lh = l // 2
                for q in range(EMBED_DIM // LANES):
                    sl = pl.ds(q * LANES, LANES)
                    rows[a][k, sl] = rows[a][k, sl] + pos_v[lh, pl.ds(ph + q * LANES, LANES)]
                return c

            lax.fori_loop(0, CHUNK, add_pos, 0, unroll=4)
            pltpu.async_copy(
                rows[a], out_hbm.at[pl.ds(base + t * CHUNK, CHUNK)], so[a]
            )

            tn = t + 2
            an = (a + 2) % NBUF

            @pl.when(tn < NCHUNK)
            def _():
                @pl.when(t >= 2)
                def _():
                    wait_out(an)

                fire_gather(tn, an)

        return carry

    lax.fori_loop(0, NCHUNK // NBUF, outer, 0)

    for a in range(NBUF):
        wait_out(a)


def kernel(x, token_emb_table):
    pos2 = _pos_encoding().reshape(SEQ_LEN // 2, 128)
    table_p = jnp.pad(token_emb_table, ((0, 0), (0, 128 - EMBED_DIM)))
    x_r = x.astype(jnp.int32).reshape(NW, NCHUNK, CHUNK)
    out = _emb_kernel(x_r, table_p, pos2)
    return out[:, :EMBED_DIM].reshape(BATCH, SEQ_LEN, EMBED_DIM)
